# baseline (device time: 97722 ns/iter reference)
import jax
import jax.numpy as jnp
from jax import lax
from jax.experimental import pallas as pl
from jax.experimental.pallas import tpu as pltpu

N_DEV = 16
H = N_DEV // 2
S = 8

RING = [0, 1, 5, 9, 13, 14, 10, 6, 2, 3, 7, 11, 15, 12, 8, 4]
POS = [RING.index(m) for m in range(N_DEV)]


def kernel(x, w_mat, scale_x, scale_w):
    m_per, k = x.shape
    n = w_mat.shape[1]
    seg = m_per // S

    def body(x_ref, w_ref, sx_ref, sw_ref, ring_ref, pos_ref, out_ref,
             cr, cl, r_send, r_recv, l_send, l_recv):
        my = lax.axis_index("i")

        p = pos_ref[my]
        right = ring_ref[lax.rem(p + 1, N_DEV)]
        left = ring_ref[lax.rem(p + N_DEV - 1, N_DEV)]

        barrier_sem = pltpu.get_barrier_semaphore()
        for nbr in (left, right):
            pl.semaphore_signal(
                barrier_sem, inc=1,
                device_id=(nbr,), device_id_type=pl.DeviceIdType.MESH,
            )
        pl.semaphore_wait(barrier_sem, 2)

        scale = sx_ref[0] * sw_ref[0]

        def gemm(chunk, o):
            acc = lax.dot_general(
                chunk, w_ref[...],
                (((1,), (0,)), ((), ())),
                preferred_element_type=jnp.int32,
            )
            y = acc.astype(jnp.float32) * scale
            out_ref[pl.ds(o * m_per, m_per), :] = y / (1.0 + jnp.exp(-y))

        rr, rl = {}, {}
        for h in range(1, H + 1):
            for s in range(S):
                if h < H or s < S // 2:
                    src = x_ref if h == 1 else cr.at[h - 1]
                    rr[(h, s)] = pltpu.make_async_remote_copy(
                        src_ref=src.at[pl.ds(s * seg, seg)],
                        dst_ref=cr.at[h, pl.ds(s * seg, seg)],
                        send_sem=r_send.at[h - 1, s],
                        recv_sem=r_recv.at[h - 1, s],
                        device_id=(right,),
                        device_id_type=pl.DeviceIdType.MESH,
                    )
                if h < H or s >= S // 2:
                    src = x_ref if h == 1 else cl.at[h - 1]
                    dst = cr if h == H else cl
                    rl[(h, s)] = pltpu.make_async_remote_copy(
                        src_ref=src.at[pl.ds(s * seg, seg)],
                        dst_ref=dst.at[h, pl.ds(s * seg, seg)],
                        send_sem=l_send.at[h - 1, s],
                        recv_sem=l_recv.at[h - 1, s],
                        device_id=(left,),
                        device_id_type=pl.DeviceIdType.MESH,
                    )

        for s in range(S):
            rr[(1, s)].start()
            rl[(1, s)].start()
        gemm(x_ref[...], my)

        for h in range(2, H + 1):
            for s in range(S):
                rr[(h - 1, s)].wait_recv()
                if (h, s) in rr:
                    rr[(h, s)].start()
                rl[(h - 1, s)].wait_recv()
                if (h, s) in rl:
                    rl[(h, s)].start()
            gemm(cr[h - 1], ring_ref[lax.rem(p + N_DEV - (h - 1), N_DEV)])
            gemm(cl[h - 1], ring_ref[lax.rem(p + h - 1, N_DEV)])

        for s in range(S // 2):
            rr[(H, s)].wait_recv()
        for s in range(S // 2, S):
            rl[(H, s)].wait_recv()
        gemm(cr[H], ring_ref[lax.rem(p + H, N_DEV)])

        for d in list(rr.values()) + list(rl.values()):
            d.wait_send()

    return pl.pallas_call(
        body,
        out_shape=jax.ShapeDtypeStruct((N_DEV * m_per, n), jnp.float32),
        in_specs=[
            pl.BlockSpec(memory_space=pltpu.VMEM),
            pl.BlockSpec(memory_space=pltpu.VMEM),
            pl.BlockSpec(memory_space=pltpu.SMEM),
            pl.BlockSpec(memory_space=pltpu.SMEM),
            pl.BlockSpec(memory_space=pltpu.SMEM),
            pl.BlockSpec(memory_space=pltpu.SMEM),
        ],
        out_specs=pl.BlockSpec(memory_space=pltpu.VMEM),
        scratch_shapes=[
            pltpu.VMEM((H + 1, m_per, k), jnp.int8),
            pltpu.VMEM((H + 1, m_per, k), jnp.int8),
            pltpu.SemaphoreType.DMA((H, S)),
            pltpu.SemaphoreType.DMA((H, S)),
            pltpu.SemaphoreType.DMA((H, S)),
            pltpu.SemaphoreType.DMA((H, S)),
        ],
        compiler_params=pltpu.CompilerParams(collective_id=0),
    )(x, w_mat, scale_x, scale_w,
      jnp.array(RING, jnp.int32), jnp.array(POS, jnp.int32))


# device time: 96572 ns/iter; 1.0119x vs baseline; 1.0119x over previous
import jax
import jax.numpy as jnp
from jax import lax
from jax.experimental import pallas as pl
from jax.experimental.pallas import tpu as pltpu

N_DEV = 16
H = N_DEV // 2
S = 4

RING = [0, 1, 5, 9, 13, 14, 10, 6, 2, 3, 7, 11, 15, 12, 8, 4]
POS = [RING.index(m) for m in range(N_DEV)]


def kernel(x, w_mat, scale_x, scale_w):
    m_per, k = x.shape
    n = w_mat.shape[1]
    seg = m_per // S

    def body(x_ref, w_ref, sx_ref, sw_ref, ring_ref, pos_ref, out_ref,
             cr, cl, r_send, r_recv, l_send, l_recv):
        my = lax.axis_index("i")

        p = pos_ref[my]
        right = ring_ref[lax.rem(p + 1, N_DEV)]
        left = ring_ref[lax.rem(p + N_DEV - 1, N_DEV)]

        barrier_sem = pltpu.get_barrier_semaphore()
        for nbr in (left, right):
            pl.semaphore_signal(
                barrier_sem, inc=1,
                device_id=(nbr,), device_id_type=pl.DeviceIdType.MESH,
            )
        pl.semaphore_wait(barrier_sem, 2)

        scale = sx_ref[0] * sw_ref[0]

        def gemm(chunk, o):
            acc = lax.dot_general(
                chunk, w_ref[...],
                (((1,), (0,)), ((), ())),
                preferred_element_type=jnp.int32,
            )
            y = acc.astype(jnp.float32) * scale
            out_ref[pl.ds(o * m_per, m_per), :] = y / (1.0 + jnp.exp(-y))

        rr, rl = {}, {}
        for h in range(1, H + 1):
            for s in range(S):
                if h < H or s < S // 2:
                    src = x_ref if h == 1 else cr.at[h - 1]
                    rr[(h, s)] = pltpu.make_async_remote_copy(
                        src_ref=src.at[pl.ds(s * seg, seg)],
                        dst_ref=cr.at[h, pl.ds(s * seg, seg)],
                        send_sem=r_send.at[h - 1, s],
                        recv_sem=r_recv.at[h - 1, s],
                        device_id=(right,),
                        device_id_type=pl.DeviceIdType.MESH,
                    )
                if h < H or s >= S // 2:
                    src = x_ref if h == 1 else cl.at[h - 1]
                    dst = cr if h == H else cl
                    rl[(h, s)] = pltpu.make_async_remote_copy(
                        src_ref=src.at[pl.ds(s * seg, seg)],
                        dst_ref=dst.at[h, pl.ds(s * seg, seg)],
                        send_sem=l_send.at[h - 1, s],
                        recv_sem=l_recv.at[h - 1, s],
                        device_id=(left,),
                        device_id_type=pl.DeviceIdType.MESH,
                    )

        for s in range(S):
            rr[(1, s)].start()
            rl[(1, s)].start()
        gemm(x_ref[...], my)

        for h in range(2, H + 1):
            for s in range(S):
                rr[(h - 1, s)].wait_recv()
                if (h, s) in rr:
                    rr[(h, s)].start()
                rl[(h - 1, s)].wait_recv()
                if (h, s) in rl:
                    rl[(h, s)].start()
            gemm(cr[h - 1], ring_ref[lax.rem(p + N_DEV - (h - 1), N_DEV)])
            gemm(cl[h - 1], ring_ref[lax.rem(p + h - 1, N_DEV)])

        for s in range(S // 2):
            rr[(H, s)].wait_recv()
        for s in range(S // 2, S):
            rl[(H, s)].wait_recv()
        gemm(cr[H], ring_ref[lax.rem(p + H, N_DEV)])

        for d in list(rr.values()) + list(rl.values()):
            d.wait_send()

    return pl.pallas_call(
        body,
        out_shape=jax.ShapeDtypeStruct((N_DEV * m_per, n), jnp.float32),
        in_specs=[
            pl.BlockSpec(memory_space=pltpu.VMEM),
            pl.BlockSpec(memory_space=pltpu.VMEM),
            pl.BlockSpec(memory_space=pltpu.SMEM),
            pl.BlockSpec(memory_space=pltpu.SMEM),
            pl.BlockSpec(memory_space=pltpu.SMEM),
            pl.BlockSpec(memory_space=pltpu.SMEM),
        ],
        out_specs=pl.BlockSpec(memory_space=pltpu.VMEM),
        scratch_shapes=[
            pltpu.VMEM((H + 1, m_per, k), jnp.int8),
            pltpu.VMEM((H + 1, m_per, k), jnp.int8),
            pltpu.SemaphoreType.DMA((H, S)),
            pltpu.SemaphoreType.DMA((H, S)),
            pltpu.SemaphoreType.DMA((H, S)),
            pltpu.SemaphoreType.DMA((H, S)),
        ],
        compiler_params=pltpu.CompilerParams(collective_id=0),
    )(x, w_mat, scale_x, scale_w,
      jnp.array(RING, jnp.int32), jnp.array(POS, jnp.int32))
